# baseline (speedup 1.0000x reference)
import functools

import jax
import jax.numpy as jnp
from jax import lax
from jax.experimental import pallas as pl
from jax.experimental.pallas import tpu as pltpu
from jax.experimental.pallas import tpu_sc as plsc

_N_GROUPS = 4
_TOPK_GROUPS = 2
_TOPK = 2
_ROUTE_SCALE = 1.0
_N_EXPERTS = 8
_TILE = 256

_NC = 2
_NS = 16
_NW = _NC * _NS
_L = 16


def _score_kernel(x_ref, w_ref, b_ref, s_ref):
    xb = x_ref[...].astype(jnp.bfloat16)
    wb = w_ref[...].astype(jnp.bfloat16)
    scores_t = jax.lax.dot_general(
        wb, xb, (((1,), (1,)), ((), ())),
        preferred_element_type=jnp.float32)
    s_ref[...] = jax.nn.sigmoid(scores_t) + b_ref[...]


def _route_kernel(tokens, s_hbm, wout_hbm, iout_hbm,
                  sbuf, w1buf, w2buf, i1buf, i2buf):
    per_w = tokens // _NW
    wid = lax.axis_index("s") * _NC + lax.axis_index("c")
    base = wid * per_w
    pltpu.sync_copy(s_hbm.at[:, pl.ds(base, per_w)], sbuf)

    e = _N_EXPERTS
    neg_inf = jnp.full((_L,), -jnp.inf, jnp.float32)
    fzero = jnp.zeros((_L,), jnp.float32)

    def body(j, carry):
        off = j * _L
        s = [sbuf[k, pl.ds(off, _L)] for k in range(e)]

        p = [s[2 * g] + s[2 * g + 1] for g in range(_N_GROUPS)]

        m1 = jnp.maximum(jnp.maximum(p[0], p[1]), jnp.maximum(p[2], p[3]))
        g1 = jnp.full((_L,), _N_GROUPS - 1, jnp.int32)
        for g in range(_N_GROUPS - 2, -1, -1):
            g1 = jnp.where(p[g] == m1, jnp.full((_L,), g, jnp.int32), g1)
        pm = [jnp.where(g1 == jnp.full((_L,), g, jnp.int32), neg_inf, p[g])
              for g in range(_N_GROUPS)]
        m2 = jnp.maximum(jnp.maximum(pm[0], pm[1]), jnp.maximum(pm[2], pm[3]))
        g2 = jnp.full((_L,), _N_GROUPS - 1, jnp.int32)
        for g in range(_N_GROUPS - 2, -1, -1):
            g2 = jnp.where(pm[g] == m2, jnp.full((_L,), g, jnp.int32), g2)

        sm = []
        for k in range(e):
            gk = jnp.full((_L,), k // (e // _N_GROUPS), jnp.int32)
            sel = (g1 == gk) | (g2 == gk)
            sm.append(jnp.where(sel, s[k], fzero))

        m1e = sm[0]
        for k in range(1, e):
            m1e = jnp.maximum(m1e, sm[k])
        i1 = jnp.full((_L,), e - 1, jnp.int32)
        for k in range(e - 2, -1, -1):
            i1 = jnp.where(sm[k] == m1e, jnp.full((_L,), k, jnp.int32), i1)
        sm2 = [jnp.where(i1 == jnp.full((_L,), k, jnp.int32), neg_inf, sm[k])
               for k in range(e)]
        m2e = sm2[0]
        for k in range(1, e):
            m2e = jnp.maximum(m2e, sm2[k])
        i2 = jnp.full((_L,), e - 1, jnp.int32)
        for k in range(e - 2, -1, -1):
            i2 = jnp.where(sm2[k] == m2e, jnp.full((_L,), k, jnp.int32), i2)

        w1 = s[e - 1]
        w2 = s[e - 1]
        for k in range(e - 2, -1, -1):
            ik = jnp.full((_L,), k, jnp.int32)
            w1 = jnp.where(i1 == ik, s[k], w1)
            w2 = jnp.where(i2 == ik, s[k], w2)
        denom = w1 + w2
        scale = jnp.full((_L,), _ROUTE_SCALE, jnp.float32)
        w1buf[pl.ds(off, _L)] = w1 / denom * scale
        w2buf[pl.ds(off, _L)] = w2 / denom * scale
        i1buf[pl.ds(off, _L)] = i1
        i2buf[pl.ds(off, _L)] = i2
        return carry

    lax.fori_loop(0, per_w // _L, body, 0)
    pltpu.sync_copy(w1buf, wout_hbm.at[0, pl.ds(base, per_w)])
    pltpu.sync_copy(w2buf, wout_hbm.at[1, pl.ds(base, per_w)])
    pltpu.sync_copy(i1buf, iout_hbm.at[0, pl.ds(base, per_w)])
    pltpu.sync_copy(i2buf, iout_hbm.at[1, pl.ds(base, per_w)])


@jax.jit
def kernel(x, weight, bias):
    tokens, dim = x.shape
    e = weight.shape[0]
    scores_t = pl.pallas_call(
        _score_kernel,
        grid=(tokens // _TILE,),
        in_specs=[
            pl.BlockSpec((_TILE, dim), lambda i: (i, 0)),
            pl.BlockSpec((e, dim), lambda i: (0, 0)),
            pl.BlockSpec((e, 1), lambda i: (0, 0)),
        ],
        out_specs=pl.BlockSpec((e, _TILE), lambda i: (0, i)),
        out_shape=jax.ShapeDtypeStruct((e, tokens), jnp.float32),
    )(x, weight, bias.reshape(e, 1))

    per_w = tokens // _NW
    mesh = plsc.VectorSubcoreMesh(core_axis_name="c", subcore_axis_name="s")
    route = pl.kernel(
        functools.partial(_route_kernel, tokens),
        out_type=[
            jax.ShapeDtypeStruct((_TOPK, tokens), jnp.float32),
            jax.ShapeDtypeStruct((_TOPK, tokens), jnp.int32),
        ],
        mesh=mesh,
        scratch_types=[
            pltpu.VMEM((e, per_w), jnp.float32),
            pltpu.VMEM((per_w,), jnp.float32),
            pltpu.VMEM((per_w,), jnp.float32),
            pltpu.VMEM((per_w,), jnp.int32),
            pltpu.VMEM((per_w,), jnp.int32),
        ],
    )
    wout_t, iout_t = route(scores_t)
    return wout_t.T.astype(x.dtype), iout_t.T


# speedup vs baseline: 1.1643x; 1.1643x over previous
import functools

import jax
import jax.numpy as jnp
from jax import lax
from jax.experimental import pallas as pl
from jax.experimental.pallas import tpu as pltpu
from jax.experimental.pallas import tpu_sc as plsc

_N_GROUPS = 4
_TOPK_GROUPS = 2
_TOPK = 2
_ROUTE_SCALE = 1.0
_N_EXPERTS = 8
_TILE = 512

_NC = 2
_NS = 16
_NW = _NC * _NS
_L = 16


def _score_kernel(x_ref, w_ref, b_ref, s_ref):
    xb = x_ref[...].astype(jnp.bfloat16)
    wb = w_ref[...].astype(jnp.bfloat16)
    scores_t = jax.lax.dot_general(
        wb, xb, (((1,), (1,)), ((), ())),
        preferred_element_type=jnp.float32)
    s_ref[...] = jax.nn.sigmoid(scores_t) + b_ref[...]


def _route_kernel(tokens, s_hbm, wout_hbm, iout_hbm,
                  sbuf, w1buf, w2buf, i1buf, i2buf):
    per_w = tokens // _NW
    wid = lax.axis_index("s") * _NC + lax.axis_index("c")
    base = wid * per_w
    pltpu.sync_copy(s_hbm.at[:, pl.ds(base, per_w)], sbuf)

    e = _N_EXPERTS
    neg_inf = jnp.full((_L,), -jnp.inf, jnp.float32)
    fzero = jnp.zeros((_L,), jnp.float32)

    def body(j, carry):
        off = j * _L
        s = [sbuf[k, pl.ds(off, _L)] for k in range(e)]

        p = [s[2 * g] + s[2 * g + 1] for g in range(_N_GROUPS)]

        m1 = jnp.maximum(jnp.maximum(p[0], p[1]), jnp.maximum(p[2], p[3]))
        g1 = jnp.full((_L,), _N_GROUPS - 1, jnp.int32)
        for g in range(_N_GROUPS - 2, -1, -1):
            g1 = jnp.where(p[g] == m1, jnp.full((_L,), g, jnp.int32), g1)
        pm = [jnp.where(g1 == jnp.full((_L,), g, jnp.int32), neg_inf, p[g])
              for g in range(_N_GROUPS)]
        m2 = jnp.maximum(jnp.maximum(pm[0], pm[1]), jnp.maximum(pm[2], pm[3]))
        g2 = jnp.full((_L,), _N_GROUPS - 1, jnp.int32)
        for g in range(_N_GROUPS - 2, -1, -1):
            g2 = jnp.where(pm[g] == m2, jnp.full((_L,), g, jnp.int32), g2)

        sm = []
        for k in range(e):
            gk = jnp.full((_L,), k // (e // _N_GROUPS), jnp.int32)
            sel = (g1 == gk) | (g2 == gk)
            sm.append(jnp.where(sel, s[k], fzero))

        m1e = sm[0]
        for k in range(1, e):
            m1e = jnp.maximum(m1e, sm[k])
        i1 = jnp.full((_L,), e - 1, jnp.int32)
        for k in range(e - 2, -1, -1):
            i1 = jnp.where(sm[k] == m1e, jnp.full((_L,), k, jnp.int32), i1)
        sm2 = [jnp.where(i1 == jnp.full((_L,), k, jnp.int32), neg_inf, sm[k])
               for k in range(e)]
        m2e = sm2[0]
        for k in range(1, e):
            m2e = jnp.maximum(m2e, sm2[k])
        i2 = jnp.full((_L,), e - 1, jnp.int32)
        for k in range(e - 2, -1, -1):
            i2 = jnp.where(sm2[k] == m2e, jnp.full((_L,), k, jnp.int32), i2)

        w1 = s[e - 1]
        w2 = s[e - 1]
        for k in range(e - 2, -1, -1):
            ik = jnp.full((_L,), k, jnp.int32)
            w1 = jnp.where(i1 == ik, s[k], w1)
            w2 = jnp.where(i2 == ik, s[k], w2)
        denom = w1 + w2
        scale = jnp.full((_L,), _ROUTE_SCALE, jnp.float32)
        w1buf[pl.ds(off, _L)] = w1 / denom * scale
        w2buf[pl.ds(off, _L)] = w2 / denom * scale
        i1buf[pl.ds(off, _L)] = i1
        i2buf[pl.ds(off, _L)] = i2
        return carry

    lax.fori_loop(0, per_w // _L, body, 0)
    pltpu.sync_copy(w1buf, wout_hbm.at[0, pl.ds(base, per_w)])
    pltpu.sync_copy(w2buf, wout_hbm.at[1, pl.ds(base, per_w)])
    pltpu.sync_copy(i1buf, iout_hbm.at[0, pl.ds(base, per_w)])
    pltpu.sync_copy(i2buf, iout_hbm.at[1, pl.ds(base, per_w)])


@jax.jit
def kernel(x, weight, bias):
    tokens, dim = x.shape
    e = weight.shape[0]
    scores_t = pl.pallas_call(
        _score_kernel,
        grid=(tokens // _TILE,),
        in_specs=[
            pl.BlockSpec((_TILE, dim), lambda i: (i, 0)),
            pl.BlockSpec((e, dim), lambda i: (0, 0)),
            pl.BlockSpec((e, 1), lambda i: (0, 0)),
        ],
        out_specs=pl.BlockSpec((e, _TILE), lambda i: (0, i)),
        out_shape=jax.ShapeDtypeStruct((e, tokens), jnp.float32),
    )(x, weight, bias.reshape(e, 1))

    per_w = tokens // _NW
    mesh = plsc.VectorSubcoreMesh(core_axis_name="c", subcore_axis_name="s")
    route = pl.kernel(
        functools.partial(_route_kernel, tokens),
        out_type=[
            jax.ShapeDtypeStruct((_TOPK, tokens), jnp.float32),
            jax.ShapeDtypeStruct((_TOPK, tokens), jnp.int32),
        ],
        mesh=mesh,
        scratch_types=[
            pltpu.VMEM((e, per_w), jnp.float32),
            pltpu.VMEM((per_w,), jnp.float32),
            pltpu.VMEM((per_w,), jnp.float32),
            pltpu.VMEM((per_w,), jnp.int32),
            pltpu.VMEM((per_w,), jnp.int32),
        ],
    )
    wout_t, iout_t = route(scores_t)
    return wout_t.T.astype(x.dtype), iout_t.T
